# baseline (speedup 1.0000x reference)
import dataclasses
import functools

import jax
import jax.numpy as jnp
from jax import lax
from jax.experimental import pallas as pl
from jax.experimental.pallas import tpu as pltpu
from jax.experimental.pallas import tpu_sc as plsc

N = 10000
E = 160000
D = 256
R = 8
NB = 4
H = 128
XB = 1000
NI = N // XB
CHUNK = 64
NTILES = 16
CPT = 160
W = 40
DEPTH = 4
E_PAD = NTILES * CPT * CHUNK
NPAD = 10112
ROWS_PER_TILE = NPAD // NTILES


def _w_body(coef_ref, bases_ref, w_ref):
    r = pl.program_id(0)
    acc = coef_ref[r, 0] * bases_ref[0]
    for b in range(1, NB):
        acc += coef_ref[r, b] * bases_ref[b]
    w_ref[0] = acc


def _make_w(coef, bases):
    return pl.pallas_call(
        _w_body,
        grid=(R,),
        in_specs=[
            pl.BlockSpec(memory_space=pltpu.SMEM),
            pl.BlockSpec((NB, D, D), lambda r: (0, 0, 0)),
        ],
        out_specs=pl.BlockSpec((1, D, D), lambda r: (r, 0, 0)),
        out_shape=jax.ShapeDtypeStruct((R, D, D), jnp.float32),
    )(coef, bases)


def _project_body(x_ref, w_ref, proj_ref):
    proj_ref[...] = jnp.dot(x_ref[...].astype(jnp.bfloat16),
                            w_ref[0].astype(jnp.bfloat16),
                            preferred_element_type=jnp.float32)


def _project(x, w):
    return pl.pallas_call(
        _project_body,
        grid=(NI, R),
        in_specs=[
            pl.BlockSpec((XB, D), lambda i, r: (i, 0)),
            pl.BlockSpec((1, D, D), lambda i, r: (r, 0, 0)),
        ],
        out_specs=pl.BlockSpec((XB, D), lambda i, r: (r * NI + i, 0)),
        out_shape=jax.ShapeDtypeStruct((R * N, D), jnp.float32),
    )(x, w)


def _selfloop_body(x_ref, lw_ref, b_ref, self_ref):
    self_ref[...] = (jnp.dot(x_ref[...].astype(jnp.bfloat16),
                             lw_ref[...].astype(jnp.bfloat16),
                             preferred_element_type=jnp.float32)
                     + b_ref[...])


def _selfloop(x, loop_w, bias2):
    return pl.pallas_call(
        _selfloop_body,
        grid=(NI,),
        in_specs=[
            pl.BlockSpec((XB, D), lambda i: (i, 0)),
            pl.BlockSpec((D, D), lambda i: (0, 0)),
            pl.BlockSpec((1, D), lambda i: (0, 0)),
        ],
        out_specs=pl.BlockSpec((XB, D), lambda i: (i, 0)),
        out_shape=jax.ShapeDtypeStruct((N, D), jnp.float32),
    )(x, loop_w, bias2)


def _combine_project_body(agg_ref, s0_ref, w_ref, lw_ref, b_ref,
                          proj_ref, self_ref, h_s):
    r = pl.program_id(1)

    @pl.when(r == 0)
    def _():
        a = jnp.concatenate([agg_ref[0], agg_ref[1]], axis=-1)
        hv = jnp.maximum(a + s0_ref[...], 0.0)
        h_s[...] = hv.astype(jnp.bfloat16)
        self_ref[...] = (jnp.dot(hv.astype(jnp.bfloat16),
                                 lw_ref[...].astype(jnp.bfloat16),
                                 preferred_element_type=jnp.float32)
                         + b_ref[...])

    proj_ref[...] = jnp.dot(h_s[...], w_ref[0].astype(jnp.bfloat16),
                            preferred_element_type=jnp.float32)


def _combine_project(agg, self0, w, loop_w, bias2):
    return pl.pallas_call(
        _combine_project_body,
        grid=(NI, R),
        in_specs=[
            pl.BlockSpec((2, XB, H), lambda i, r: (0, i, 0)),
            pl.BlockSpec((XB, D), lambda i, r: (i, 0)),
            pl.BlockSpec((1, D, D), lambda i, r: (r, 0, 0)),
            pl.BlockSpec((D, D), lambda i, r: (0, 0)),
            pl.BlockSpec((1, D), lambda i, r: (0, 0)),
        ],
        out_specs=[
            pl.BlockSpec((XB, D), lambda i, r: (r * NI + i, 0)),
            pl.BlockSpec((XB, D), lambda i, r: (i, 0)),
        ],
        out_shape=[
            jax.ShapeDtypeStruct((R * N, D), jnp.float32),
            jax.ShapeDtypeStruct((N, D), jnp.float32),
        ],
        scratch_shapes=[pltpu.VMEM((XB, D), jnp.bfloat16)],
    )(agg, self0, w, loop_w, bias2)


def _final_body(agg_ref, s1_ref, out_ref):
    out_ref[...] = (jnp.concatenate([agg_ref[0], agg_ref[1]], axis=-1)
                    + s1_ref[...])


def _final(agg, self1):
    return pl.pallas_call(
        _final_body,
        grid=(NI,),
        in_specs=[
            pl.BlockSpec((2, XB, H), lambda i: (0, i, 0)),
            pl.BlockSpec((XB, D), lambda i: (i, 0)),
        ],
        out_specs=pl.BlockSpec((XB, D), lambda i: (i, 0)),
        out_shape=jax.ShapeDtypeStruct((N, D), jnp.float32),
    )(agg, self1)


def _sc_body(proj_hbm, idx2_hbm, dst_hbm, norm_hbm, out_hbm,
             acc_sh, idx_a, dst_a, norm_a,
             rows0, rows1, rows2, rows3,
             g0, g1, g2, g3, s0, s1, s2, s3):
    c = lax.axis_index("c")
    s = lax.axis_index("s")
    rows = [rows0, rows1, rows2, rows3]
    gsem = [g0, g1, g2, g3]
    ssem = [s0, s1, s2, s3]

    @pl.loop(0, CHUNK)
    def _(i):
        for j in range(H // 16):
            rows0[i, pl.ds(j * 16, 16)] = jnp.zeros((16,), jnp.float32)

    base = s * ROWS_PER_TILE
    for b in range(9):
        pltpu.sync_copy(rows0, acc_sh.at[pl.ds(base + b * CHUNK, CHUNK)])
    pltpu.sync_copy(rows0.at[pl.ds(0, 56)],
                    acc_sh.at[pl.ds(base + 9 * CHUNK, 56)])
    plsc.subcore_barrier()

    def scale(buf, j):
        @plsc.parallel_loop(0, CHUNK, unroll=4)
        def _(i):
            nb = plsc.load_gather(
                norm_a, [jnp.full((16,), j, jnp.int32),
                         jnp.full((16,), i, jnp.int32)])
            for jj in range(H // 16):
                sl = pl.ds(jj * 16, 16)
                buf[i, sl] = buf[i, sl] * nb

    for win in range(CPT // W):
        woff = s * CPT + win * W
        pltpu.sync_copy(idx2_hbm.at[c, pl.ds(woff, W)], idx_a)
        pltpu.sync_copy(dst_hbm.at[pl.ds(woff, W)], dst_a)
        pltpu.sync_copy(norm_hbm.at[pl.ds(woff, W)], norm_a)

        for b in range(DEPTH):
            pltpu.async_copy(proj_hbm.at[idx_a.at[b]], rows[b], gsem[b])

        @pl.loop(0, W // DEPTH)
        def _(q):
            jq = DEPTH * q
            for b in range(DEPTH):
                j = jq + b
                pltpu.make_async_copy(proj_hbm.at[idx_a.at[0]],
                                      rows[b], gsem[b]).wait()
                scale(rows[b], j)
                pltpu.async_copy(rows[b], acc_sh.at[dst_a.at[j]],
                                 add=True, sem=ssem[b])

                @pl.when(q < W // DEPTH - 1)
                def _():
                    pltpu.make_async_copy(rows[b], acc_sh.at[dst_a.at[0]],
                                          ssem[b]).wait()
                    pltpu.async_copy(proj_hbm.at[idx_a.at[j + DEPTH]],
                                     rows[b], gsem[b])

        for b in range(DEPTH):
            pltpu.make_async_copy(rows[b], acc_sh.at[dst_a.at[0]],
                                  ssem[b]).wait()

    plsc.subcore_barrier()
    for b in range(9):
        pltpu.sync_copy(acc_sh.at[pl.ds(base + b * CHUNK, CHUNK)],
                        out_hbm.at[c, pl.ds(base + b * CHUNK, CHUNK)])
    pltpu.sync_copy(acc_sh.at[pl.ds(base + 9 * CHUNK, 56)],
                    out_hbm.at[c, pl.ds(base + 9 * CHUNK, 56)])


def _sc_aggregate(proj, idx2, dst, norm):
    mesh = plsc.VectorSubcoreMesh(core_axis_name="c", subcore_axis_name="s")
    cp = pltpu.CompilerParams()
    if "needs_layout_passes" in pltpu.CompilerParams.__dataclass_fields__:
        cp = dataclasses.replace(cp, needs_layout_passes=False)
    f = pl.kernel(
        _sc_body,
        out_type=jax.ShapeDtypeStruct((2, NPAD, H), jnp.float32),
        mesh=mesh,
        scratch_types=[
            pltpu.VMEM_SHARED((NPAD, H), jnp.float32),
            pltpu.VMEM((W, CHUNK), jnp.int32),
            pltpu.VMEM((W, CHUNK), jnp.int32),
            pltpu.VMEM((W, CHUNK), jnp.float32),
            pltpu.VMEM((CHUNK, H), jnp.float32),
            pltpu.VMEM((CHUNK, H), jnp.float32),
            pltpu.VMEM((CHUNK, H), jnp.float32),
            pltpu.VMEM((CHUNK, H), jnp.float32),
            pltpu.SemaphoreType.DMA,
            pltpu.SemaphoreType.DMA,
            pltpu.SemaphoreType.DMA,
            pltpu.SemaphoreType.DMA,
            pltpu.SemaphoreType.DMA,
            pltpu.SemaphoreType.DMA,
            pltpu.SemaphoreType.DMA,
            pltpu.SemaphoreType.DMA,
        ],
        compiler_params=cp,
    )
    return f(proj, idx2, dst, norm)


def kernel(emb, edge_index, etypes, edge_norm, bases0, coef0, loop_w0, bias0,
           bases1, coef1, loop_w1, bias1):
    src = edge_index[0].astype(jnp.int32)
    dst = edge_index[1].astype(jnp.int32)
    et = etypes.astype(jnp.int32)
    norm = edge_norm.reshape(-1).astype(jnp.float32)

    pad = E_PAD - E
    flat = et * N + src
    flat = jnp.concatenate([flat, jnp.zeros((pad,), jnp.int32)])
    dst = jnp.concatenate([dst, jnp.full((pad,), N, jnp.int32)])
    norm = jnp.concatenate([norm, jnp.zeros((pad,), jnp.float32)])
    idx2 = jnp.stack([2 * flat, 2 * flat + 1]).reshape(2, -1, CHUNK)
    dst = dst.reshape(-1, CHUNK)
    norm = norm.reshape(-1, CHUNK)

    w0 = _make_w(coef0, bases0)
    proj0 = _project(emb, w0)
    self0 = _selfloop(emb, loop_w0, bias0.reshape(1, D))
    agg0 = _sc_aggregate(proj0.reshape(2 * R * N, H), idx2, dst, norm)

    w1 = _make_w(coef1, bases1)
    proj1, self1 = _combine_project(agg0, self0, w1, loop_w1,
                                    bias1.reshape(1, D))
    agg1 = _sc_aggregate(proj1.reshape(2 * R * N, H), idx2, dst, norm)

    return _final(agg1, self1)


# speedup vs baseline: 1.0146x; 1.0146x over previous
import dataclasses
import functools

import jax
import jax.numpy as jnp
from jax import lax
from jax.experimental import pallas as pl
from jax.experimental.pallas import tpu as pltpu
from jax.experimental.pallas import tpu_sc as plsc

N = 10000
E = 160000
D = 256
R = 8
NB = 4
H = 128
XB = 1000
NI = N // XB
CHUNK = 64
NTILES = 16
CPT = 160
W = 40
DEPTH = 4
E_PAD = NTILES * CPT * CHUNK
NPAD = 10112
ROWS_PER_TILE = NPAD // NTILES


def _w_body(coef_ref, bases_ref, w_ref):
    r = pl.program_id(0)
    acc = coef_ref[r, 0] * bases_ref[0]
    for b in range(1, NB):
        acc += coef_ref[r, b] * bases_ref[b]
    w_ref[0] = acc


def _make_w(coef, bases):
    return pl.pallas_call(
        _w_body,
        grid=(R,),
        in_specs=[
            pl.BlockSpec(memory_space=pltpu.SMEM),
            pl.BlockSpec((NB, D, D), lambda r: (0, 0, 0)),
        ],
        out_specs=pl.BlockSpec((1, D, D), lambda r: (r, 0, 0)),
        out_shape=jax.ShapeDtypeStruct((R, D, D), jnp.float32),
    )(coef, bases)


def _project_body(x_ref, w_ref, proj_ref):
    proj_ref[...] = jnp.dot(x_ref[...].astype(jnp.bfloat16),
                            w_ref[0, 0].astype(jnp.bfloat16),
                            preferred_element_type=jnp.float32)


def _project(x, w4):
    return pl.pallas_call(
        _project_body,
        grid=(NI, R, 2),
        in_specs=[
            pl.BlockSpec((XB, D), lambda i, r, h: (i, 0)),
            pl.BlockSpec((1, 1, D, H), lambda i, r, h: (h, r, 0, 0)),
        ],
        out_specs=pl.BlockSpec((XB, H),
                               lambda i, r, h: (h * (R * NI) + r * NI + i, 0)),
        out_shape=jax.ShapeDtypeStruct((2 * R * N, H), jnp.float32),
    )(x, w4)


def _selfloop_body(x_ref, lw_ref, b_ref, self_ref):
    self_ref[...] = (jnp.dot(x_ref[...].astype(jnp.bfloat16),
                             lw_ref[...].astype(jnp.bfloat16),
                             preferred_element_type=jnp.float32)
                     + b_ref[...])


def _selfloop(x, loop_w, bias2):
    return pl.pallas_call(
        _selfloop_body,
        grid=(NI,),
        in_specs=[
            pl.BlockSpec((XB, D), lambda i: (i, 0)),
            pl.BlockSpec((D, D), lambda i: (0, 0)),
            pl.BlockSpec((1, D), lambda i: (0, 0)),
        ],
        out_specs=pl.BlockSpec((XB, D), lambda i: (i, 0)),
        out_shape=jax.ShapeDtypeStruct((N, D), jnp.float32),
    )(x, loop_w, bias2)


def _combine_project_body(agg_ref, s0_ref, w_ref, lw_ref, b_ref,
                          proj_ref, self_ref, h_s):
    r = pl.program_id(1)
    h = pl.program_id(2)

    @pl.when((r == 0) & (h == 0))
    def _():
        a = jnp.concatenate([agg_ref[0], agg_ref[1]], axis=-1)
        hv = jnp.maximum(a + s0_ref[...], 0.0)
        h_s[...] = hv.astype(jnp.bfloat16)
        self_ref[...] = (jnp.dot(hv.astype(jnp.bfloat16),
                                 lw_ref[...].astype(jnp.bfloat16),
                                 preferred_element_type=jnp.float32)
                         + b_ref[...])

    proj_ref[...] = jnp.dot(h_s[...], w_ref[0, 0].astype(jnp.bfloat16),
                            preferred_element_type=jnp.float32)


def _combine_project(agg, self0, w4, loop_w, bias2):
    return pl.pallas_call(
        _combine_project_body,
        grid=(NI, R, 2),
        in_specs=[
            pl.BlockSpec((2, XB, H), lambda i, r, h: (0, i, 0)),
            pl.BlockSpec((XB, D), lambda i, r, h: (i, 0)),
            pl.BlockSpec((1, 1, D, H), lambda i, r, h: (h, r, 0, 0)),
            pl.BlockSpec((D, D), lambda i, r, h: (0, 0)),
            pl.BlockSpec((1, D), lambda i, r, h: (0, 0)),
        ],
        out_specs=[
            pl.BlockSpec((XB, H), lambda i, r, h: (h * (R * NI) + r * NI + i, 0)),
            pl.BlockSpec((XB, D), lambda i, r, h: (i, 0)),
        ],
        out_shape=[
            jax.ShapeDtypeStruct((2 * R * N, H), jnp.float32),
            jax.ShapeDtypeStruct((N, D), jnp.float32),
        ],
        scratch_shapes=[pltpu.VMEM((XB, D), jnp.bfloat16)],
    )(agg, self0, w4, loop_w, bias2)


def _final_body(agg_ref, s1_ref, out_ref):
    out_ref[...] = (jnp.concatenate([agg_ref[0], agg_ref[1]], axis=-1)
                    + s1_ref[...])


def _final(agg, self1):
    return pl.pallas_call(
        _final_body,
        grid=(NI,),
        in_specs=[
            pl.BlockSpec((2, XB, H), lambda i: (0, i, 0)),
            pl.BlockSpec((XB, D), lambda i: (i, 0)),
        ],
        out_specs=pl.BlockSpec((XB, D), lambda i: (i, 0)),
        out_shape=jax.ShapeDtypeStruct((N, D), jnp.float32),
    )(agg, self1)


def _sc_body(proj_hbm, idx2_hbm, dst_hbm, norm_hbm, out_hbm,
             acc_sh, idx_a, dst_a, norm_a,
             rows0, rows1, rows2, rows3,
             g0, g1, g2, g3, s0, s1, s2, s3):
    c = lax.axis_index("c")
    s = lax.axis_index("s")
    rows = [rows0, rows1, rows2, rows3]
    gsem = [g0, g1, g2, g3]
    ssem = [s0, s1, s2, s3]

    @pl.loop(0, CHUNK)
    def _(i):
        for j in range(H // 16):
            rows0[i, pl.ds(j * 16, 16)] = jnp.zeros((16,), jnp.float32)

    base = s * ROWS_PER_TILE
    for b in range(9):
        pltpu.sync_copy(rows0, acc_sh.at[pl.ds(base + b * CHUNK, CHUNK)])
    pltpu.sync_copy(rows0.at[pl.ds(0, 56)],
                    acc_sh.at[pl.ds(base + 9 * CHUNK, 56)])
    plsc.subcore_barrier()

    def scale(buf, j):
        @plsc.parallel_loop(0, CHUNK, unroll=4)
        def _(i):
            nb = plsc.load_gather(
                norm_a, [jnp.full((16,), j, jnp.int32),
                         jnp.full((16,), i, jnp.int32)])
            for jj in range(H // 16):
                sl = pl.ds(jj * 16, 16)
                buf[i, sl] = buf[i, sl] * nb

    for win in range(CPT // W):
        woff = s * CPT + win * W
        pltpu.sync_copy(idx2_hbm.at[c, pl.ds(woff, W)], idx_a)
        pltpu.sync_copy(dst_hbm.at[pl.ds(woff, W)], dst_a)
        pltpu.sync_copy(norm_hbm.at[pl.ds(woff, W)], norm_a)

        for b in range(DEPTH):
            pltpu.async_copy(proj_hbm.at[idx_a.at[b]], rows[b], gsem[b])

        @pl.loop(0, W // DEPTH)
        def _(q):
            jq = DEPTH * q
            for b in range(DEPTH):
                j = jq + b
                pltpu.make_async_copy(proj_hbm.at[idx_a.at[0]],
                                      rows[b], gsem[b]).wait()
                scale(rows[b], j)
                pltpu.async_copy(rows[b], acc_sh.at[dst_a.at[j]],
                                 add=True, sem=ssem[b])

                @pl.when(q < W // DEPTH - 1)
                def _():
                    pltpu.make_async_copy(rows[b], acc_sh.at[dst_a.at[0]],
                                          ssem[b]).wait()
                    pltpu.async_copy(proj_hbm.at[idx_a.at[j + DEPTH]],
                                     rows[b], gsem[b])

        for b in range(DEPTH):
            pltpu.make_async_copy(rows[b], acc_sh.at[dst_a.at[0]],
                                  ssem[b]).wait()

    plsc.subcore_barrier()
    for b in range(9):
        pltpu.sync_copy(acc_sh.at[pl.ds(base + b * CHUNK, CHUNK)],
                        out_hbm.at[c, pl.ds(base + b * CHUNK, CHUNK)])
    pltpu.sync_copy(acc_sh.at[pl.ds(base + 9 * CHUNK, 56)],
                    out_hbm.at[c, pl.ds(base + 9 * CHUNK, 56)])


def _sc_aggregate(proj, idx2, dst, norm):
    mesh = plsc.VectorSubcoreMesh(core_axis_name="c", subcore_axis_name="s")
    cp = pltpu.CompilerParams()
    if "needs_layout_passes" in pltpu.CompilerParams.__dataclass_fields__:
        cp = dataclasses.replace(cp, needs_layout_passes=False)
    f = pl.kernel(
        _sc_body,
        out_type=jax.ShapeDtypeStruct((2, NPAD, H), jnp.float32),
        mesh=mesh,
        scratch_types=[
            pltpu.VMEM_SHARED((NPAD, H), jnp.float32),
            pltpu.VMEM((W, CHUNK), jnp.int32),
            pltpu.VMEM((W, CHUNK), jnp.int32),
            pltpu.VMEM((W, CHUNK), jnp.float32),
            pltpu.VMEM((CHUNK, H), jnp.float32),
            pltpu.VMEM((CHUNK, H), jnp.float32),
            pltpu.VMEM((CHUNK, H), jnp.float32),
            pltpu.VMEM((CHUNK, H), jnp.float32),
            pltpu.SemaphoreType.DMA,
            pltpu.SemaphoreType.DMA,
            pltpu.SemaphoreType.DMA,
            pltpu.SemaphoreType.DMA,
            pltpu.SemaphoreType.DMA,
            pltpu.SemaphoreType.DMA,
            pltpu.SemaphoreType.DMA,
            pltpu.SemaphoreType.DMA,
        ],
        compiler_params=cp,
    )
    return f(proj, idx2, dst, norm)


def kernel(emb, edge_index, etypes, edge_norm, bases0, coef0, loop_w0, bias0,
           bases1, coef1, loop_w1, bias1):
    src = edge_index[0].astype(jnp.int32)
    dst = edge_index[1].astype(jnp.int32)
    et = etypes.astype(jnp.int32)
    norm = edge_norm.reshape(-1).astype(jnp.float32)

    pad = E_PAD - E
    flat = et * N + src
    flat = jnp.concatenate([flat, jnp.zeros((pad,), jnp.int32)])
    dst = jnp.concatenate([dst, jnp.full((pad,), N, jnp.int32)])
    norm = jnp.concatenate([norm, jnp.zeros((pad,), jnp.float32)])
    idx2 = jnp.stack([flat, flat + R * N]).reshape(2, -1, CHUNK)
    dst = dst.reshape(-1, CHUNK)
    norm = norm.reshape(-1, CHUNK)

    w0 = jnp.moveaxis(_make_w(coef0, bases0).reshape(R, D, 2, H), 2, 0)
    proj0 = _project(emb, w0)
    self0 = _selfloop(emb, loop_w0, bias0.reshape(1, D))
    agg0 = _sc_aggregate(proj0, idx2, dst, norm)

    w1 = jnp.moveaxis(_make_w(coef1, bases1).reshape(R, D, 2, H), 2, 0)
    proj1, self1 = _combine_project(agg0, self0, w1, loop_w1,
                                    bias1.reshape(1, D))
    agg1 = _sc_aggregate(proj1, idx2, dst, norm)

    return _final(agg1, self1)


# speedup vs baseline: 1.0701x; 1.0547x over previous
import dataclasses
import functools

import jax
import jax.numpy as jnp
from jax import lax
from jax.experimental import pallas as pl
from jax.experimental.pallas import tpu as pltpu
from jax.experimental.pallas import tpu_sc as plsc

N = 10000
E = 160000
D = 256
R = 8
NB = 4
H = 128
XB = 1000
NI = N // XB
CHUNK = 64
NTILES = 16
CPT = 160
W = 40
DEPTH = 4
E_PAD = NTILES * CPT * CHUNK
NPAD = 10112
ROWS_PER_TILE = NPAD // NTILES


def _w_body(coef_ref, bases_ref, w_ref):
    r = pl.program_id(0)
    acc = coef_ref[r, 0] * bases_ref[0]
    for b in range(1, NB):
        acc += coef_ref[r, b] * bases_ref[b]
    w_ref[0] = acc


def _make_w(coef, bases):
    return pl.pallas_call(
        _w_body,
        grid=(R,),
        in_specs=[
            pl.BlockSpec(memory_space=pltpu.SMEM),
            pl.BlockSpec((NB, D, D), lambda r: (0, 0, 0)),
        ],
        out_specs=pl.BlockSpec((1, D, D), lambda r: (r, 0, 0)),
        out_shape=jax.ShapeDtypeStruct((R, D, D), jnp.float32),
    )(coef, bases)


def _project_body(x_ref, w_ref, lw_ref, b_ref, proj_ref, self_ref):
    r = pl.program_id(1)
    h = pl.program_id(2)
    proj_ref[...] = jnp.dot(x_ref[...], w_ref[0, 0],
                            preferred_element_type=jnp.float32)

    @pl.when((r == 0) & (h == 0))
    def _():
        self_ref[...] = (jnp.dot(x_ref[...], lw_ref[...],
                                 preferred_element_type=jnp.float32)
                         + b_ref[...])


def _project(x, w4, loop_w, bias2):
    return pl.pallas_call(
        _project_body,
        grid=(NI, R, 2),
        in_specs=[
            pl.BlockSpec((XB, D), lambda i, r, h: (i, 0)),
            pl.BlockSpec((1, 1, D, H), lambda i, r, h: (h, r, 0, 0)),
            pl.BlockSpec((D, D), lambda i, r, h: (0, 0)),
            pl.BlockSpec((1, D), lambda i, r, h: (0, 0)),
        ],
        out_specs=[
            pl.BlockSpec((XB, H), lambda i, r, h: (h * (R * NI) + r * NI + i, 0)),
            pl.BlockSpec((XB, D), lambda i, r, h: (i, 0)),
        ],
        out_shape=[
            jax.ShapeDtypeStruct((2 * R * N, H), jnp.float32),
            jax.ShapeDtypeStruct((N, D), jnp.float32),
        ],
    )(x, w4, loop_w, bias2)


def _combine_project_body(agg_ref, s0_ref, w_ref, lw_ref, b_ref,
                          proj_ref, self_ref, h_s):
    r = pl.program_id(1)
    h = pl.program_id(2)

    @pl.when((r == 0) & (h == 0))
    def _():
        a = jnp.concatenate([agg_ref[0], agg_ref[1]], axis=-1)
        hv = jnp.maximum(a + s0_ref[...], 0.0)
        h_s[...] = hv
        self_ref[...] = (jnp.dot(hv, lw_ref[...],
                                 preferred_element_type=jnp.float32)
                         + b_ref[...])

    proj_ref[...] = jnp.dot(h_s[...], w_ref[0, 0],
                            preferred_element_type=jnp.float32)


def _combine_project(agg, self0, w4, loop_w, bias2):
    return pl.pallas_call(
        _combine_project_body,
        grid=(NI, R, 2),
        in_specs=[
            pl.BlockSpec((2, XB, H), lambda i, r, h: (0, i, 0)),
            pl.BlockSpec((XB, D), lambda i, r, h: (i, 0)),
            pl.BlockSpec((1, 1, D, H), lambda i, r, h: (h, r, 0, 0)),
            pl.BlockSpec((D, D), lambda i, r, h: (0, 0)),
            pl.BlockSpec((1, D), lambda i, r, h: (0, 0)),
        ],
        out_specs=[
            pl.BlockSpec((XB, H), lambda i, r, h: (h * (R * NI) + r * NI + i, 0)),
            pl.BlockSpec((XB, D), lambda i, r, h: (i, 0)),
        ],
        out_shape=[
            jax.ShapeDtypeStruct((2 * R * N, H), jnp.float32),
            jax.ShapeDtypeStruct((N, D), jnp.float32),
        ],
        scratch_shapes=[pltpu.VMEM((XB, D), jnp.float32)],
    )(agg, self0, w4, loop_w, bias2)


def _final_body(agg_ref, s1_ref, out_ref):
    out_ref[...] = (jnp.concatenate([agg_ref[0], agg_ref[1]], axis=-1)
                    + s1_ref[...])


def _final(agg, self1):
    return pl.pallas_call(
        _final_body,
        grid=(NI,),
        in_specs=[
            pl.BlockSpec((2, XB, H), lambda i: (0, i, 0)),
            pl.BlockSpec((XB, D), lambda i: (i, 0)),
        ],
        out_specs=pl.BlockSpec((XB, D), lambda i: (i, 0)),
        out_shape=jax.ShapeDtypeStruct((N, D), jnp.float32),
    )(agg, self1)


def _sc_body(proj_hbm, idx2_hbm, dst_hbm, norm_hbm, out_hbm,
             acc_sh, idx_a, dst_a, norm_a,
             rows0, rows1, rows2, rows3,
             g0, g1, g2, g3, s0, s1, s2, s3):
    c = lax.axis_index("c")
    s = lax.axis_index("s")
    rows = [rows0, rows1, rows2, rows3]
    gsem = [g0, g1, g2, g3]
    ssem = [s0, s1, s2, s3]

    @pl.loop(0, CHUNK)
    def _(i):
        for j in range(H // 16):
            rows0[i, pl.ds(j * 16, 16)] = jnp.zeros((16,), jnp.float32)

    base = s * ROWS_PER_TILE
    for b in range(9):
        pltpu.sync_copy(rows0, acc_sh.at[pl.ds(base + b * CHUNK, CHUNK)])
    pltpu.sync_copy(rows0.at[pl.ds(0, 56)],
                    acc_sh.at[pl.ds(base + 9 * CHUNK, 56)])
    plsc.subcore_barrier()

    def scale(buf, j):
        @plsc.parallel_loop(0, CHUNK, unroll=4)
        def _(i):
            nb = plsc.load_gather(
                norm_a, [jnp.full((16,), j, jnp.int32),
                         jnp.full((16,), i, jnp.int32)])
            for jj in range(H // 16):
                sl = pl.ds(jj * 16, 16)
                buf[i, sl] = buf[i, sl] * nb

    for win in range(CPT // W):
        woff = s * CPT + win * W
        pltpu.sync_copy(idx2_hbm.at[c, pl.ds(woff, W)], idx_a)
        pltpu.sync_copy(dst_hbm.at[pl.ds(woff, W)], dst_a)
        pltpu.sync_copy(norm_hbm.at[pl.ds(woff, W)], norm_a)

        for b in range(DEPTH):
            pltpu.async_copy(proj_hbm.at[idx_a.at[b]], rows[b], gsem[b])

        @pl.loop(0, W // DEPTH)
        def _(q):
            jq = DEPTH * q
            for b in range(DEPTH):
                j = jq + b
                pltpu.make_async_copy(proj_hbm.at[idx_a.at[0]],
                                      rows[b], gsem[b]).wait()
                scale(rows[b], j)
                pltpu.async_copy(rows[b], acc_sh.at[dst_a.at[j]],
                                 add=True, sem=ssem[b])

                @pl.when(q < W // DEPTH - 1)
                def _():
                    pltpu.make_async_copy(rows[b], acc_sh.at[dst_a.at[0]],
                                          ssem[b]).wait()
                    pltpu.async_copy(proj_hbm.at[idx_a.at[j + DEPTH]],
                                     rows[b], gsem[b])

        for b in range(DEPTH):
            pltpu.make_async_copy(rows[b], acc_sh.at[dst_a.at[0]],
                                  ssem[b]).wait()

    plsc.subcore_barrier()
    for b in range(9):
        pltpu.sync_copy(acc_sh.at[pl.ds(base + b * CHUNK, CHUNK)],
                        out_hbm.at[c, pl.ds(base + b * CHUNK, CHUNK)])
    pltpu.sync_copy(acc_sh.at[pl.ds(base + 9 * CHUNK, 56)],
                    out_hbm.at[c, pl.ds(base + 9 * CHUNK, 56)])


def _sc_aggregate(proj, idx2, dst, norm):
    mesh = plsc.VectorSubcoreMesh(core_axis_name="c", subcore_axis_name="s")
    cp = pltpu.CompilerParams()
    if "needs_layout_passes" in pltpu.CompilerParams.__dataclass_fields__:
        cp = dataclasses.replace(cp, needs_layout_passes=False)
    f = pl.kernel(
        _sc_body,
        out_type=jax.ShapeDtypeStruct((2, NPAD, H), jnp.float32),
        mesh=mesh,
        scratch_types=[
            pltpu.VMEM_SHARED((NPAD, H), jnp.float32),
            pltpu.VMEM((W, CHUNK), jnp.int32),
            pltpu.VMEM((W, CHUNK), jnp.int32),
            pltpu.VMEM((W, CHUNK), jnp.float32),
            pltpu.VMEM((CHUNK, H), jnp.float32),
            pltpu.VMEM((CHUNK, H), jnp.float32),
            pltpu.VMEM((CHUNK, H), jnp.float32),
            pltpu.VMEM((CHUNK, H), jnp.float32),
            pltpu.SemaphoreType.DMA,
            pltpu.SemaphoreType.DMA,
            pltpu.SemaphoreType.DMA,
            pltpu.SemaphoreType.DMA,
            pltpu.SemaphoreType.DMA,
            pltpu.SemaphoreType.DMA,
            pltpu.SemaphoreType.DMA,
            pltpu.SemaphoreType.DMA,
        ],
        compiler_params=cp,
    )
    return f(proj, idx2, dst, norm)


def kernel(emb, edge_index, etypes, edge_norm, bases0, coef0, loop_w0, bias0,
           bases1, coef1, loop_w1, bias1):
    src = edge_index[0].astype(jnp.int32)
    dst = edge_index[1].astype(jnp.int32)
    et = etypes.astype(jnp.int32)
    norm = edge_norm.reshape(-1).astype(jnp.float32)

    pad = E_PAD - E
    flat = et * N + src
    flat = jnp.concatenate([flat, jnp.zeros((pad,), jnp.int32)])
    dst = jnp.concatenate([dst, jnp.full((pad,), N, jnp.int32)])
    norm = jnp.concatenate([norm, jnp.zeros((pad,), jnp.float32)])
    idx2 = jnp.stack([flat, flat + R * N]).reshape(2, -1, CHUNK)
    dst = dst.reshape(-1, CHUNK)
    norm = norm.reshape(-1, CHUNK)

    w0 = jnp.moveaxis(_make_w(coef0, bases0).reshape(R, D, 2, H), 2, 0)
    proj0, self0 = _project(emb, w0, loop_w0, bias0.reshape(1, D))
    agg0 = _sc_aggregate(proj0, idx2, dst, norm)

    w1 = jnp.moveaxis(_make_w(coef1, bases1).reshape(R, D, 2, H), 2, 0)
    proj1, self1 = _combine_project(agg0, self0, w1, loop_w1,
                                    bias1.reshape(1, D))
    agg1 = _sc_aggregate(proj1, idx2, dst, norm)

    return _final(agg1, self1)
